# Initial kernel scaffold; baseline (speedup 1.0000x reference)
import jax
import jax.numpy as jnp
from jax.experimental import pallas as pl


def kernel(x, noise_tm, wi, si, wrec, wo, so, h0):
    raise NotImplementedError("write your pallas kernel here")


# speedup vs baseline: 2.3660x; 2.3660x over previous
import functools

import jax
import jax.numpy as jnp
from jax.experimental import pallas as pl
from jax.experimental.pallas import tpu as pltpu

_ALPHA = 0.2
_NOISE_STD = 0.05


def _round_up(x, m):
    return ((x + m - 1) // m) * m


def _rnn_kernel(x_ref, noise_ref, wi_ref, wrec_ref, wo_ref, h0_ref, out_ref,
                h_c, r_c, drive_ref, rate_ref, *, bc, ts):
    H = wrec_ref.shape[0]

    @pl.when(pl.program_id(1) == 0)
    def _init():
        h0b = jnp.broadcast_to(h0_ref[...], (bc, H))
        h_c[...] = h0b
        r_c[...] = jnp.tanh(h0b).astype(jnp.bfloat16)

    x2d = x_ref[...].reshape(ts * bc, x_ref.shape[-1])
    inp = jnp.dot(x2d, wi_ref[...], preferred_element_type=jnp.float32)
    n2d = noise_ref[...].reshape(ts * bc, H)
    drive_ref[...] = _NOISE_STD * n2d + inp

    wrec = wrec_ref[...]

    def step(j, carry):
        h, r = carry
        off = pl.multiple_of(j * bc, bc)
        rec = jnp.dot(r, wrec, preferred_element_type=jnp.float32)
        h_new = (1.0 - _ALPHA) * h + drive_ref[pl.ds(off, bc), :] + rec
        r_new = jnp.tanh(h_new).astype(jnp.bfloat16)
        rate_ref[pl.ds(off, bc), :] = r_new
        return (h_new, r_new)

    h_fin, r_fin = jax.lax.fori_loop(0, ts, step, (h_c[...], r_c[...]),
                                     unroll=8)
    h_c[...] = h_fin
    r_c[...] = r_fin

    out = jnp.dot(rate_ref[...], wo_ref[...],
                  preferred_element_type=jnp.float32)
    out_ref[...] = out.reshape(ts, bc, out.shape[-1])


def kernel(x, noise_tm, wi, si, wrec, wo, so, h0):
    B, S, I = x.shape
    H = wrec.shape[0]
    O = wo.shape[1]

    wi_a = (_ALPHA * (wi * si[:, None])).astype(jnp.bfloat16)
    wrec_ta = (_ALPHA * jnp.transpose(wrec)).astype(jnp.bfloat16)
    wo_full = (wo * so[None, :]).astype(jnp.bfloat16)
    h0_2d = h0.reshape(1, H).astype(jnp.float32)

    BC = _round_up(max(1, (B + 1) // 2), 8)
    Bp = 2 * BC

    TS = min(64, S)
    S_pad = _round_up(S, TS)
    NT = S_pad // TS

    x_p = jnp.pad(x, ((0, Bp - B), (0, S_pad - S), (0, 0)))
    x_tm = jnp.transpose(x_p.reshape(2, BC, S_pad, I), (0, 2, 1, 3))
    x2d = x_tm.reshape(2, S_pad * BC, I).astype(jnp.bfloat16)

    n_p = jnp.pad(noise_tm, ((0, S_pad - S), (0, Bp - B), (0, 0)))

    _kernel_fn = functools.partial(_rnn_kernel, bc=BC, ts=TS)

    grid_spec = pltpu.PrefetchScalarGridSpec(
        num_scalar_prefetch=0,
        grid=(2, NT),
        in_specs=[
            pl.BlockSpec((1, TS * BC, I), lambda c, t: (c, t, 0)),
            pl.BlockSpec((TS, BC, H), lambda c, t: (t, c, 0)),
            pl.BlockSpec((I, H), lambda c, t: (0, 0)),
            pl.BlockSpec((H, H), lambda c, t: (0, 0)),
            pl.BlockSpec((H, O), lambda c, t: (0, 0)),
            pl.BlockSpec((1, H), lambda c, t: (0, 0)),
        ],
        out_specs=pl.BlockSpec((TS, BC, O), lambda c, t: (t, c, 0)),
        scratch_shapes=[
            pltpu.VMEM((BC, H), jnp.float32),
            pltpu.VMEM((BC, H), jnp.bfloat16),
            pltpu.VMEM((TS * BC, H), jnp.float32),
            pltpu.VMEM((TS * BC, H), jnp.bfloat16),
        ],
    )

    out_tm = pl.pallas_call(
        _kernel_fn,
        out_shape=jax.ShapeDtypeStruct((S_pad, Bp, O), jnp.float32),
        grid_spec=grid_spec,
        compiler_params=pltpu.CompilerParams(
            dimension_semantics=("parallel", "arbitrary"),
            vmem_limit_bytes=30 * 2**20),
    )(x2d, n_p, wi_a, wrec_ta, wo_full, h0_2d)

    out = out_tm[:S, :B]
    return jnp.transpose(out, (1, 0, 2))


# speedup vs baseline: 3.4123x; 1.4422x over previous
import functools

import jax
import jax.numpy as jnp
from jax.experimental import pallas as pl
from jax.experimental.pallas import tpu as pltpu

_ALPHA = 0.2
_NOISE_STD = 0.05


def _round_up(x, m):
    return ((x + m - 1) // m) * m


def _rnn_kernel(x_ref, noise_ref, wi_ref, wrec_ref, wo_ref, h0_ref, out_ref,
                h_c, r_c, drive_ref, rate_ref, *, bp, ts):
    H = wrec_ref.shape[0]

    @pl.when(pl.program_id(0) == 0)
    def _init():
        h0b = jnp.broadcast_to(h0_ref[...], (bp, H))
        h_c[...] = h0b
        r_c[...] = jnp.tanh(h0b).astype(jnp.bfloat16)

    inp = jnp.dot(x_ref[...], wi_ref[...], preferred_element_type=jnp.float32)
    n2d = noise_ref[...].reshape(ts * bp, H)
    drive_ref[...] = _NOISE_STD * n2d + inp

    wrec = wrec_ref[...]

    def step(j, carry):
        h, r = carry
        off = pl.multiple_of(j * bp, bp)
        rec = jnp.dot(r, wrec, preferred_element_type=jnp.float32)
        h_new = (1.0 - _ALPHA) * h + drive_ref[pl.ds(off, bp), :] + rec
        r_new = jnp.tanh(h_new).astype(jnp.bfloat16)
        rate_ref[pl.ds(off, bp), :] = r_new
        return (h_new, r_new)

    h_fin, r_fin = jax.lax.fori_loop(0, ts, step, (h_c[...], r_c[...]),
                                     unroll=8)
    h_c[...] = h_fin
    r_c[...] = r_fin

    out = jnp.dot(rate_ref[...], wo_ref[...],
                  preferred_element_type=jnp.float32)
    out_ref[...] = out.reshape(ts, bp, out.shape[-1])


def kernel(x, noise_tm, wi, si, wrec, wo, so, h0):
    B, S, I = x.shape
    H = wrec.shape[0]
    O = wo.shape[1]

    wi_a = (_ALPHA * (wi * si[:, None])).astype(jnp.bfloat16)
    wrec_ta = (_ALPHA * jnp.transpose(wrec)).astype(jnp.bfloat16)
    wo_full = (wo * so[None, :]).astype(jnp.bfloat16)
    h0_2d = h0.reshape(1, H).astype(jnp.float32)

    Bp = _round_up(max(B, 8), 8)

    TS = min(64, S)
    S_pad = _round_up(S, TS)
    NT = S_pad // TS

    x_p = jnp.pad(x, ((0, Bp - B), (0, S_pad - S), (0, 0)))
    x2d = jnp.transpose(x_p, (1, 0, 2)).reshape(S_pad * Bp, I)
    x2d = x2d.astype(jnp.bfloat16)

    n_p = jnp.pad(noise_tm, ((0, S_pad - S), (0, Bp - B), (0, 0)))

    _kernel_fn = functools.partial(_rnn_kernel, bp=Bp, ts=TS)

    grid_spec = pltpu.PrefetchScalarGridSpec(
        num_scalar_prefetch=0,
        grid=(NT,),
        in_specs=[
            pl.BlockSpec((TS * Bp, I), lambda t: (t, 0)),
            pl.BlockSpec((TS, Bp, H), lambda t: (t, 0, 0)),
            pl.BlockSpec((I, H), lambda t: (0, 0)),
            pl.BlockSpec((H, H), lambda t: (0, 0)),
            pl.BlockSpec((H, O), lambda t: (0, 0)),
            pl.BlockSpec((1, H), lambda t: (0, 0)),
        ],
        out_specs=pl.BlockSpec((TS, Bp, O), lambda t: (t, 0, 0)),
        scratch_shapes=[
            pltpu.VMEM((Bp, H), jnp.float32),
            pltpu.VMEM((Bp, H), jnp.bfloat16),
            pltpu.VMEM((TS * Bp, H), jnp.float32),
            pltpu.VMEM((TS * Bp, H), jnp.bfloat16),
        ],
    )

    out_tm = pl.pallas_call(
        _kernel_fn,
        out_shape=jax.ShapeDtypeStruct((S_pad, Bp, O), jnp.float32),
        grid_spec=grid_spec,
        compiler_params=pltpu.CompilerParams(
            dimension_semantics=("arbitrary",),
            vmem_limit_bytes=48 * 2**20),
    )(x2d, n_p, wi_a, wrec_ta, wo_full, h0_2d)

    out = out_tm[:S, :B]
    return jnp.transpose(out, (1, 0, 2))
